# baseline (speedup 1.0000x reference)
import jax
import jax.numpy as jnp
from jax import lax
from jax.experimental import pallas as pl
from jax.experimental.pallas import tpu as pltpu
from jax.experimental.pallas import tpu_sc as plsc

B = 16384
D = 64
LANES = 16
NUM_WORKERS = 32
BPW = B // NUM_WORKERS
GCHUNK = 128
NCHUNKS = BPW // GCHUNK


def _body(users_hbm, items_hbm, utab_hbm, itab_hbm, out_hbm,
          idx_u, idx_v, rows_u, rows_v, out_v, sem):
  wid = lax.axis_index("s") * 2 + lax.axis_index("c")
  base = wid * BPW

  pltpu.sync_copy(users_hbm.at[pl.ds(base, BPW)], idx_u)
  pltpu.sync_copy(items_hbm.at[pl.ds(base, BPW)], idx_v)

  copies = []
  for j in range(NCHUNKS):
    sl = pl.ds(j * GCHUNK, GCHUNK)
    copies.append(pltpu.make_async_copy(
        utab_hbm.at[idx_u.at[sl]], rows_u.at[sl], sem))
    copies.append(pltpu.make_async_copy(
        itab_hbm.at[idx_v.at[sl]], rows_v.at[sl], sem))
  for c in copies:
    c.start()
  for c in copies:
    c.wait()

  lane = lax.iota(jnp.int32, LANES)

  def group_body(g, _):
    ridx = g * LANES + lane
    acc = jnp.zeros((LANES,), jnp.float32)
    for d in range(D):
      cidx = jnp.full((LANES,), d, jnp.int32)
      acc = acc + (plsc.load_gather(rows_u, [ridx, cidx])
                   * plsc.load_gather(rows_v, [ridx, cidx]))
    out_v[pl.ds(g * LANES, LANES)] = 1.0 / (1.0 + jnp.exp(-acc))
    return 0

  lax.fori_loop(0, BPW // LANES, group_body, 0)

  pltpu.sync_copy(out_v, out_hbm.at[pl.ds(base, BPW)])


@jax.jit
def kernel(users, items, user_table, item_table):
  mesh = plsc.VectorSubcoreMesh(core_axis_name="c", subcore_axis_name="s")
  run = pl.kernel(
      _body,
      out_type=jax.ShapeDtypeStruct((B,), jnp.float32),
      mesh=mesh,
      scratch_types=[
          pltpu.VMEM((BPW,), jnp.int32),
          pltpu.VMEM((BPW,), jnp.int32),
          pltpu.VMEM((BPW, D), jnp.float32),
          pltpu.VMEM((BPW, D), jnp.float32),
          pltpu.VMEM((BPW,), jnp.float32),
          pltpu.SemaphoreType.DMA,
      ],
      compiler_params=pltpu.CompilerParams(
          use_tc_tiling_on_sc=False,
          needs_layout_passes=False,
      ),
  )
  return run(users, items, user_table, item_table)
